# Initial kernel scaffold; baseline (speedup 1.0000x reference)
import functools

import jax
import jax.numpy as jnp
from jax import lax
from jax.experimental import pallas as pl
from jax.experimental.pallas import tpu as pltpu
from jax.experimental.pallas import tpu_sc as plsc

NN = 100000
HD = 16
NE = 1600000
NT = 16
B = 512
IDXW = 128
NSUB = B // IDXW
EPT = 100352
EPAD = NT * EPT
RPT = EPT // IDXW
NCHUNK = EPT // B
ZR = NN // NT
ZFULL = ZR // B
ZREM = ZR - ZFULL * B

_mesh = plsc.VectorSubcoreMesh(core_axis_name="c", subcore_axis_name="s")


def _layer_body(emb, srcr, dstr, vals, out, sidx, didx, vbuf, rows, acc, sem):
    c = lax.axis_index("c")
    s = lax.axis_index("s")
    cN = c * NN

    @pl.loop(0, B)
    def _(i):
        rows[i] = jnp.zeros((HD,), jnp.float32)

    zbase = s * ZR

    @pl.loop(0, ZFULL)
    def _(z):
        pltpu.sync_copy(rows.at[pl.ds(0, B)], acc.at[pl.ds(zbase + z * B, B)])

    pltpu.sync_copy(rows.at[pl.ds(0, ZREM)],
                    acc.at[pl.ds(zbase + ZFULL * B, ZREM)])
    plsc.subcore_barrier()

    rbase = s * RPT
    vbase = s * EPT

    @pl.loop(0, NCHUNK)
    def _(g):
        pltpu.sync_copy(srcr.at[pl.ds(rbase + g * NSUB, NSUB)], sidx)
        pltpu.sync_copy(dstr.at[pl.ds(rbase + g * NSUB, NSUB)], didx)
        pltpu.sync_copy(vals.at[pl.ds(vbase + g * B, B)], vbuf)

        @pl.loop(0, NSUB)
        def _(j):
            @pl.loop(0, IDXW // 16)
            def _(k):
                sl = pl.ds(k * 16, 16)
                sidx[j, sl] = sidx[j, sl] + cN

        for j in range(NSUB):
            pltpu.async_copy(emb.at[sidx.at[j]],
                             rows.at[pl.ds(j * IDXW, IDXW)], sem).wait()

        @pl.loop(0, B)
        def _(e):
            rows[e] = rows[e] * vbuf[e]

        for j in range(NSUB):
            pltpu.sync_copy(rows.at[pl.ds(j * IDXW, IDXW)],
                            acc.at[didx.at[j]], add=True)

    plsc.subcore_barrier()

    ob = cN + s * ZR

    @pl.loop(0, ZFULL)
    def _(z):
        pltpu.sync_copy(acc.at[pl.ds(zbase + z * B, B)],
                        out.at[pl.ds(ob + z * B, B)])

    pltpu.sync_copy(acc.at[pl.ds(zbase + ZFULL * B, ZREM)],
                    out.at[pl.ds(ob + ZFULL * B, ZREM)])


_layer = pl.kernel(
    _layer_body,
    out_type=jax.ShapeDtypeStruct((2 * NN, HD), jnp.float32),
    mesh=_mesh,
    scratch_types=[
        pltpu.VMEM((NSUB, IDXW), jnp.int32),
        pltpu.VMEM((NSUB, IDXW), jnp.int32),
        pltpu.VMEM((B,), jnp.float32),
        pltpu.VMEM((B, HD), jnp.float32),
        pltpu.VMEM_SHARED((NN, HD), jnp.float32),
        pltpu.SemaphoreType.DMA,
    ],
)


def _mean_body(a0, b0, a1, b1, a2, b2, a3, b3, o):
    sl = (a0[0] + a1[0] + a2[0] + a3[0]) * 0.25
    sr = (b0[0] + b1[0] + b2[0] + b3[0]) * 0.25
    o[...] = jnp.concatenate([sl, sr], axis=1)


def _mean4(e0, e1, e2, e3):
    bn = 4000
    r = lambda x: x.reshape(2, NN, HD)
    in_l = pl.BlockSpec((1, bn, HD), lambda i: (0, i, 0))
    in_r = pl.BlockSpec((1, bn, HD), lambda i: (1, i, 0))
    call = pl.pallas_call(
        _mean_body,
        grid=(NN // bn,),
        in_specs=[in_l, in_r] * 4,
        out_specs=pl.BlockSpec((bn, 2 * HD), lambda i: (i, 0)),
        out_shape=jax.ShapeDtypeStruct((NN, 2 * HD), jnp.float32),
    )
    return call(r(e0), r(e0), r(e1), r(e1), r(e2), r(e2), r(e3), r(e3))


def kernel(user_emb, item_emb, adj_indices, adj_values):
    n_users = user_emb.shape[0]
    dst = adj_indices[0].astype(jnp.int32)
    src = adj_indices[1].astype(jnp.int32)
    val = adj_values.astype(jnp.float32)

    pad = EPAD - NE
    pad_idx = (jnp.arange(pad, dtype=jnp.int32) * 17) % NN
    src_p = jnp.concatenate([src, pad_idx])
    dst_p = jnp.concatenate([dst, pad_idx])
    val_p = jnp.concatenate([val, jnp.zeros((pad,), jnp.float32)])

    srcr = src_p.reshape(EPAD // IDXW, IDXW)
    dstr = dst_p.reshape(EPAD // IDXW, IDXW)

    all_emb = jnp.concatenate([user_emb, item_emb], axis=0)
    e0 = jnp.concatenate([all_emb[:, :HD], all_emb[:, HD:]], axis=0)

    e1 = _layer(e0, srcr, dstr, val_p)
    e2 = _layer(e1, srcr, dstr, val_p)
    e3 = _layer(e2, srcr, dstr, val_p)

    final = _mean4(e0, e1, e2, e3)
    return final[:n_users], final[n_users:]


# speedup vs baseline: 6.7741x; 6.7741x over previous
import jax
import jax.numpy as jnp
from jax import lax
from jax.experimental import pallas as pl
from jax.experimental.pallas import tpu as pltpu
from jax.experimental.pallas import tpu_sc as plsc

NN = 100000
NPAD = 100096
HD = 16
NE = 1600000
NT = 16
B = 1024
IDXW = 128
NSUB = B // IDXW
EPT = 100352
EPAD = NT * EPT
RPT = EPT // IDXW
NCHUNK = EPT // B
ZR = NPAD // NT
ZFULL = ZR // 512
ZREM = ZR - ZFULL * 512

_mesh = plsc.VectorSubcoreMesh(core_axis_name="c", subcore_axis_name="s")


def _layer_body(emb, srcr, dstr, vals, out, sidx, didx, vbuf, rows, acc, sem):
    c = lax.axis_index("c")
    s = lax.axis_index("s")
    cN = c * NPAD

    @pl.loop(0, 512)
    def _(i):
        rows[i] = jnp.zeros((HD,), jnp.float32)

    zbase = s * ZR

    @pl.loop(0, ZFULL)
    def _(z):
        pltpu.sync_copy(rows.at[pl.ds(0, 512)],
                        acc.at[pl.ds(zbase + z * 512, 512)])

    pltpu.sync_copy(rows.at[pl.ds(0, ZREM)],
                    acc.at[pl.ds(zbase + ZFULL * 512, ZREM)])
    plsc.subcore_barrier()

    rbase = s * RPT
    vbase = s * EPT

    @pl.loop(0, NCHUNK)
    def _(g):
        pltpu.sync_copy(srcr.at[pl.ds(rbase + g * NSUB, NSUB)], sidx)
        pltpu.sync_copy(dstr.at[pl.ds(rbase + g * NSUB, NSUB)], didx)
        pltpu.sync_copy(vals.at[pl.ds(vbase + g * B, B)], vbuf)

        @pl.loop(0, NSUB)
        def _(j):
            @pl.loop(0, IDXW // 16)
            def _(k):
                sl = pl.ds(k * 16, 16)
                sidx[j, sl] = sidx[j, sl] + cN

        for j in range(NSUB):
            pltpu.async_copy(emb.at[sidx.at[j]],
                             rows.at[pl.ds(j * IDXW, IDXW)], sem).wait()

        @pl.loop(0, B // 16)
        def _(q):
            v16 = vbuf[pl.ds(q * 16, 16)]
            for i in range(16):
                rows[q * 16 + i] = rows[q * 16 + i] * v16[i]

        for j in range(NSUB):
            pltpu.sync_copy(rows.at[pl.ds(j * IDXW, IDXW)],
                            acc.at[didx.at[j]], add=True)

    plsc.subcore_barrier()

    ob = cN + s * ZR

    @pl.loop(0, ZFULL)
    def _(z):
        pltpu.sync_copy(acc.at[pl.ds(zbase + z * 512, 512)],
                        out.at[pl.ds(ob + z * 512, 512)])

    pltpu.sync_copy(acc.at[pl.ds(zbase + ZFULL * 512, ZREM)],
                    out.at[pl.ds(ob + ZFULL * 512, ZREM)])


_layer = pl.kernel(
    _layer_body,
    out_type=jax.ShapeDtypeStruct((2 * NPAD, HD), jnp.float32),
    mesh=_mesh,
    compiler_params=pltpu.CompilerParams(use_tc_tiling_on_sc=False),
    scratch_types=[
        pltpu.VMEM((NSUB, IDXW), jnp.int32),
        pltpu.VMEM((NSUB, IDXW), jnp.int32),
        pltpu.VMEM((B,), jnp.float32),
        pltpu.VMEM((B, HD), jnp.float32),
        pltpu.VMEM_SHARED((NPAD, HD), jnp.float32),
        pltpu.SemaphoreType.DMA,
    ],
)


def _mean_body(a0, b0, a1, b1, a2, b2, a3, b3, o):
    sl = (a0[0] + a1[0] + a2[0] + a3[0]) * 0.25
    sr = (b0[0] + b1[0] + b2[0] + b3[0]) * 0.25
    o[...] = jnp.concatenate([sl, sr], axis=1)


def _mean4(e0, e1, e2, e3):
    bn = 4000
    r = lambda x: x.reshape(2, NPAD, HD)
    in_l = pl.BlockSpec((1, bn, HD), lambda i: (0, i, 0))
    in_r = pl.BlockSpec((1, bn, HD), lambda i: (1, i, 0))
    call = pl.pallas_call(
        _mean_body,
        grid=(NN // bn,),
        in_specs=[in_l, in_r] * 4,
        out_specs=pl.BlockSpec((bn, 2 * HD), lambda i: (i, 0)),
        out_shape=jax.ShapeDtypeStruct((NN, 2 * HD), jnp.float32),
    )
    return call(r(e0), r(e0), r(e1), r(e1), r(e2), r(e2), r(e3), r(e3))


def kernel(user_emb, item_emb, adj_indices, adj_values):
    n_users = user_emb.shape[0]
    dst = adj_indices[0].astype(jnp.int32)
    src = adj_indices[1].astype(jnp.int32)
    val = adj_values.astype(jnp.float32)

    pad = EPAD - NE
    pad_idx = (jnp.arange(pad, dtype=jnp.int32) * 17) % NN
    src_p = jnp.concatenate([src, pad_idx])
    dst_p = jnp.concatenate([dst, pad_idx])
    val_p = jnp.concatenate([val, jnp.zeros((pad,), jnp.float32)])

    srcr = src_p.reshape(EPAD // IDXW, IDXW)
    dstr = dst_p.reshape(EPAD // IDXW, IDXW)

    all_emb = jnp.concatenate([user_emb, item_emb], axis=0)
    row_pad = ((0, NPAD - NN), (0, 0))
    e0 = jnp.concatenate([jnp.pad(all_emb[:, :HD], row_pad),
                          jnp.pad(all_emb[:, HD:], row_pad)], axis=0)

    e1 = _layer(e0, srcr, dstr, val_p)
    e2 = _layer(e1, srcr, dstr, val_p)
    e3 = _layer(e2, srcr, dstr, val_p)

    final = _mean4(e0, e1, e2, e3)
    return final[:n_users], final[n_users:]


# speedup vs baseline: 14.0748x; 2.0777x over previous
import jax
import jax.numpy as jnp
from jax import lax
from jax.experimental import pallas as pl
from jax.experimental.pallas import tpu as pltpu
from jax.experimental.pallas import tpu_sc as plsc

NN = 100000
NPAD = 100096
HD = 16
NE = 1600000
NT = 16
B = 512
IDXW = 128
NSUB = B // IDXW
EPT = 100352
EPAD = NT * EPT
RPT = EPT // IDXW
NCHUNK = EPT // B
ZR = NPAD // NT
ZFULL = ZR // 512
ZREM = ZR - ZFULL * 512

_mesh = plsc.VectorSubcoreMesh(core_axis_name="c", subcore_axis_name="s")


def _layer_body(emb, srcr, dstr, vals, out, sidx, didx, vbuf, rows, acc,
                semi, semg0, semg1, sems0, sems1):
    c = lax.axis_index("c")
    s = lax.axis_index("s")
    cN = c * NPAD

    @pl.loop(0, 512)
    def _(i):
        rows[0, i] = jnp.zeros((HD,), jnp.float32)

    zbase = s * ZR

    @pl.loop(0, ZFULL)
    def _(z):
        pltpu.sync_copy(rows.at[0, pl.ds(0, 512)],
                        acc.at[pl.ds(zbase + z * 512, 512)])

    pltpu.sync_copy(rows.at[0, pl.ds(0, ZREM)],
                    acc.at[pl.ds(zbase + ZFULL * 512, ZREM)])
    plsc.subcore_barrier()

    rbase = s * RPT
    vbase = s * EPT
    semg = (semg0, semg1)
    sems = (sems0, sems1)

    def idx_cps(g, b):
        r0 = rbase + g * NSUB
        return [
            pltpu.make_async_copy(srcr.at[pl.ds(r0, NSUB)], sidx.at[b], semi),
            pltpu.make_async_copy(dstr.at[pl.ds(r0, NSUB)], didx.at[b], semi),
            pltpu.make_async_copy(vals.at[pl.ds(vbase + g * B, B)],
                                  vbuf.at[b], semi),
        ]

    def gather_cps(b):
        return [
            pltpu.make_async_copy(emb.at[sidx.at[b, j]],
                                  rows.at[b, pl.ds(j * IDXW, IDXW)], semg[b])
            for j in range(NSUB)
        ]

    def scatter_cps(b):
        return [
            pltpu.make_async_copy(rows.at[b, pl.ds(j * IDXW, IDXW)],
                                  acc.at[didx.at[b, j]], sems[b])
            for j in range(NSUB)
        ]

    def prep(g, b):
        for cp in idx_cps(g, b):
            cp.start()
        for cp in idx_cps(g, b):
            cp.wait()

        @pl.loop(0, NSUB)
        def _(j):
            @pl.loop(0, IDXW // 16)
            def _(k):
                sl = pl.ds(k * 16, 16)
                sidx[b, j, sl] = sidx[b, j, sl] + cN

        for cp in gather_cps(b):
            cp.start()

    prep(0, 0)

    @pl.loop(0, NCHUNK // 2)
    def _(t):
        for b in range(2):
            g = 2 * t + b
            for cp in gather_cps(b):
                cp.wait()

            def bracket(wait_prev):
                def go():
                    if wait_prev:
                        for cp in scatter_cps(1 - b):
                            cp.wait()
                    prep(g + 1, 1 - b)
                return go

            if b == 0:
                pl.when(t > 0)(bracket(True))
                pl.when(t == 0)(bracket(False))
            else:
                pl.when(t < NCHUNK // 2 - 1)(bracket(True))

            @pl.loop(0, B // 16)
            def _(q):
                v16 = vbuf[b, pl.ds(q * 16, 16)]
                for i in range(16):
                    rows[b, q * 16 + i] = rows[b, q * 16 + i] * v16[i]

            for cp in scatter_cps(b):
                cp.start(add=True)

    for cp in scatter_cps(0):
        cp.wait()
    for cp in scatter_cps(1):
        cp.wait()
    plsc.subcore_barrier()

    ob = cN + s * ZR

    @pl.loop(0, ZFULL)
    def _(z):
        pltpu.sync_copy(acc.at[pl.ds(zbase + z * 512, 512)],
                        out.at[pl.ds(ob + z * 512, 512)])

    pltpu.sync_copy(acc.at[pl.ds(zbase + ZFULL * 512, ZREM)],
                    out.at[pl.ds(ob + ZFULL * 512, ZREM)])


_layer = pl.kernel(
    _layer_body,
    out_type=jax.ShapeDtypeStruct((2 * NPAD, HD), jnp.float32),
    mesh=_mesh,
    compiler_params=pltpu.CompilerParams(use_tc_tiling_on_sc=False),
    scratch_types=[
        pltpu.VMEM((2, NSUB, IDXW), jnp.int32),
        pltpu.VMEM((2, NSUB, IDXW), jnp.int32),
        pltpu.VMEM((2, B), jnp.float32),
        pltpu.VMEM((2, B, HD), jnp.float32),
        pltpu.VMEM_SHARED((NPAD, HD), jnp.float32),
        pltpu.SemaphoreType.DMA,
        pltpu.SemaphoreType.DMA,
        pltpu.SemaphoreType.DMA,
        pltpu.SemaphoreType.DMA,
        pltpu.SemaphoreType.DMA,
    ],
)


def _mean_body(a0, b0, a1, b1, a2, b2, a3, b3, o):
    sl = (a0[0] + a1[0] + a2[0] + a3[0]) * 0.25
    sr = (b0[0] + b1[0] + b2[0] + b3[0]) * 0.25
    o[...] = jnp.concatenate([sl, sr], axis=1)


def _mean4(e0, e1, e2, e3):
    bn = 4000
    r = lambda x: x.reshape(2, NPAD, HD)
    in_l = pl.BlockSpec((1, bn, HD), lambda i: (0, i, 0))
    in_r = pl.BlockSpec((1, bn, HD), lambda i: (1, i, 0))
    call = pl.pallas_call(
        _mean_body,
        grid=(NN // bn,),
        in_specs=[in_l, in_r] * 4,
        out_specs=pl.BlockSpec((bn, 2 * HD), lambda i: (i, 0)),
        out_shape=jax.ShapeDtypeStruct((NN, 2 * HD), jnp.float32),
    )
    return call(r(e0), r(e0), r(e1), r(e1), r(e2), r(e2), r(e3), r(e3))


def kernel(user_emb, item_emb, adj_indices, adj_values):
    n_users = user_emb.shape[0]
    dst = adj_indices[0].astype(jnp.int32)
    src = adj_indices[1].astype(jnp.int32)
    val = adj_values.astype(jnp.float32)

    pad = EPAD - NE
    pad_idx = (jnp.arange(pad, dtype=jnp.int32) * 17) % NN
    src_p = jnp.concatenate([src, pad_idx])
    dst_p = jnp.concatenate([dst, pad_idx])
    val_p = jnp.concatenate([val, jnp.zeros((pad,), jnp.float32)])

    srcr = src_p.reshape(EPAD // IDXW, IDXW)
    dstr = dst_p.reshape(EPAD // IDXW, IDXW)

    all_emb = jnp.concatenate([user_emb, item_emb], axis=0)
    row_pad = ((0, NPAD - NN), (0, 0))
    e0 = jnp.concatenate([jnp.pad(all_emb[:, :HD], row_pad),
                          jnp.pad(all_emb[:, HD:], row_pad)], axis=0)

    e1 = _layer(e0, srcr, dstr, val_p)
    e2 = _layer(e1, srcr, dstr, val_p)
    e3 = _layer(e2, srcr, dstr, val_p)

    final = _mean4(e0, e1, e2, e3)
    return final[:n_users], final[n_users:]
